# baseline (speedup 1.0000x reference)
import jax
import jax.numpy as jnp
from jax.experimental import pallas as pl

_B, _N, _C = 16, 65536, 10
_MIN_ROI = (0.0, -51.2, -5.0)
_GRID = 0.4
_PTS = 2048


def _radar_kernel(in_ref, feat_ref, idx_ref):
    pts = in_ref[...]
    feat_ref[...] = pts
    pid = pl.program_id(0)
    ids = pid * _PTS + jax.lax.broadcasted_iota(jnp.int32, (_PTS, 1), 0)
    b = ids // _N
    x_min, y_min, z_min = _MIN_ROI
    xi = jnp.ceil((pts[:, 0:1] - x_min) / _GRID).astype(jnp.int32)
    yi = jnp.ceil((pts[:, 1:2] - y_min) / _GRID).astype(jnp.int32)
    zi = jnp.ceil((pts[:, 2:3] - z_min) / _GRID).astype(jnp.int32)
    idx_ref[...] = jnp.concatenate([b, zi, yi, xi], axis=1)


def kernel(rdr_sparse_cube):
    bn = _B * _N
    flat = rdr_sparse_cube.reshape(bn, _C)
    grid = bn // _PTS
    feat, idx = pl.pallas_call(
        _radar_kernel,
        grid=(grid,),
        in_specs=[pl.BlockSpec((_PTS, _C), lambda i: (i, 0))],
        out_specs=[
            pl.BlockSpec((_PTS, _C), lambda i: (i, 0)),
            pl.BlockSpec((_PTS, 4), lambda i: (i, 0)),
        ],
        out_shape=[
            jax.ShapeDtypeStruct((bn, _C), jnp.float32),
            jax.ShapeDtypeStruct((bn, 4), jnp.int32),
        ],
    )(flat)
    return feat, idx
